# baseline (speedup 1.0000x reference)
import functools

import jax
import jax.numpy as jnp
from jax import lax
from jax.experimental import pallas as pl
from jax.experimental.pallas import tpu as pltpu
from jax.experimental.pallas import tpu_sc as plsc

B = 16
C = 3
N = 8192
M = 1024
L = 16
H = N // 2


def _fps_body(x_hbm, y_hbm, xs_v, ys_v, zs_v, xh_v, yh_v, zh_v,
              dist_v, idx_v, ox_v, oy_v, oz_v, mb_v, mi_v, pb_v, pi_v,
              sh_ex):
    cid = lax.axis_index("c")
    sid = lax.axis_index("s")
    batch = cid * 8 + sid // 2
    half = sid % 2
    off = half * H

    row = batch * C
    pltpu.sync_copy(x_hbm.at[row + 0], xs_v)
    pltpu.sync_copy(x_hbm.at[row + 1], ys_v)
    pltpu.sync_copy(x_hbm.at[row + 2], zs_v)
    pltpu.sync_copy(x_hbm.at[row + 0, pl.ds(off, H)], xh_v)
    pltpu.sync_copy(x_hbm.at[row + 1, pl.ds(off, H)], yh_v)
    pltpu.sync_copy(x_hbm.at[row + 2, pl.ds(off, H)], zh_v)

    inf16 = jnp.full((L,), jnp.inf, jnp.float32)

    def init_j(j, carry):
        dist_v[pl.ds(j * L, L)] = inf16
        return carry

    lax.fori_loop(0, H // L, init_j, 0)
    idx_v[pl.ds(0, L)] = jnp.zeros((L,), jnp.int32)

    lane = lax.iota(jnp.int32, L)

    def step(t, k):
        lx = plsc.load_gather(xs_v, [k])
        ly = plsc.load_gather(ys_v, [k])
        lz = plsc.load_gather(zs_v, [k])

        best0 = jnp.full((L,), -jnp.inf, jnp.float32)
        bidx0 = jnp.full((L,), 2**30, jnp.int32)

        def chunk(s, carry):
            best, bidx = carry
            dx = xh_v[pl.ds(s, L)] - lx
            dy = yh_v[pl.ds(s, L)] - ly
            dz = zh_v[pl.ds(s, L)] - lz
            d = (dx * dx + dz * dz) + dy * dy
            nd = jnp.minimum(dist_v[pl.ds(s, L)], d)
            dist_v[pl.ds(s, L)] = nd
            idx = off + s + lane
            gt = nd > best
            eq = nd == best
            bidx = jnp.where(gt, idx,
                             jnp.where(eq, jnp.minimum(bidx, idx), bidx))
            best = jnp.maximum(best, nd)
            return best, bidx

        best, bidx = plsc.parallel_loop(
            0, H, step=L, unroll=16, carry=(best0, bidx0))(chunk)

        prow = 300 + (t % 2) * 32
        mb_v[...] = best
        mi_v[...] = plsc.bitcast(bidx, jnp.float32)
        pltpu.sync_copy(mb_v, sh_ex.at[cid, prow + sid])
        pltpu.sync_copy(mi_v, sh_ex.at[cid, prow + L + sid])
        plsc.subcore_barrier()
        pltpu.sync_copy(sh_ex.at[cid, prow + (sid ^ 1)], pb_v)
        pltpu.sync_copy(sh_ex.at[cid, prow + L + (sid ^ 1)], pi_v)
        obest = pb_v[...]
        obidx = plsc.bitcast(pi_v[...], jnp.int32)

        gt = obest > best
        eq = obest == best
        bidx = jnp.where(gt, obidx,
                         jnp.where(eq, jnp.minimum(bidx, obidx), bidx))
        best = jnp.maximum(best, obest)

        maxv = jnp.max(best)
        cand = jnp.where(best == maxv, bidx, jnp.int32(2**30))
        knext = jnp.full((L,), jnp.min(cand), jnp.int32)

        @pl.when(half == 0)
        def _():
            plsc.store_scatter(idx_v, [jnp.full((L,), t, jnp.int32)],
                               knext, mask=lane == 0)

        return knext

    lax.fori_loop(1, M, step, jnp.zeros((L,), jnp.int32))

    @pl.when(half == 0)
    def _():
        def gout(j, carry):
            s = j * L
            iv = idx_v[pl.ds(s, L)]
            ox_v[pl.ds(s, L)] = plsc.load_gather(xs_v, [iv])
            oy_v[pl.ds(s, L)] = plsc.load_gather(ys_v, [iv])
            oz_v[pl.ds(s, L)] = plsc.load_gather(zs_v, [iv])
            return carry

        lax.fori_loop(0, M // L, gout, 0)

        pltpu.sync_copy(ox_v, y_hbm.at[row + 0])
        pltpu.sync_copy(oy_v, y_hbm.at[row + 1])
        pltpu.sync_copy(oz_v, y_hbm.at[row + 2])


@jax.jit
def _fps_sc(xr):
    mesh = plsc.VectorSubcoreMesh(core_axis_name="c", subcore_axis_name="s")
    f = functools.partial(
        pl.kernel,
        mesh=mesh,
        compiler_params=pltpu.CompilerParams(needs_layout_passes=False),
        out_type=jax.ShapeDtypeStruct((B * C, M), jnp.float32),
        scratch_types=[
            pltpu.VMEM((N,), jnp.float32),
            pltpu.VMEM((N,), jnp.float32),
            pltpu.VMEM((N,), jnp.float32),
            pltpu.VMEM((H,), jnp.float32),
            pltpu.VMEM((H,), jnp.float32),
            pltpu.VMEM((H,), jnp.float32),
            pltpu.VMEM((H,), jnp.float32),
            pltpu.VMEM((M,), jnp.int32),
            pltpu.VMEM((M,), jnp.float32),
            pltpu.VMEM((M,), jnp.float32),
            pltpu.VMEM((M,), jnp.float32),
            pltpu.VMEM((L,), jnp.float32),
            pltpu.VMEM((L,), jnp.float32),
            pltpu.VMEM((L,), jnp.float32),
            pltpu.VMEM((L,), jnp.float32),
            pltpu.VMEM_SHARED((2, 512, L), jnp.float32),
        ],
    )(_fps_body)
    return f(xr)


def kernel(x):
    xr = x.reshape(B * C, N)
    yr = _fps_sc(xr)
    return yr.reshape(B, C, M)


# speedup vs baseline: 1.0837x; 1.0837x over previous
import functools

import jax
import jax.numpy as jnp
from jax import lax
from jax.experimental import pallas as pl
from jax.experimental.pallas import tpu as pltpu
from jax.experimental.pallas import tpu_sc as plsc

B = 16
C = 3
N = 8192
M = 1024
L = 16
H = N // 2


def _fps_body(x_hbm, y_hbm, xs_v, ys_v, zs_v, xh_v, yh_v, zh_v,
              dist_v, idx_v, ox_v, oy_v, oz_v, mb_v, mi_v, pb_v, pi_v,
              sh_ex):
    cid = lax.axis_index("c")
    sid = lax.axis_index("s")
    batch = cid * 8 + sid // 2
    half = sid % 2
    off = half * H

    row = batch * C
    pltpu.sync_copy(x_hbm.at[row + 0], xs_v)
    pltpu.sync_copy(x_hbm.at[row + 1], ys_v)
    pltpu.sync_copy(x_hbm.at[row + 2], zs_v)
    pltpu.sync_copy(x_hbm.at[row + 0, pl.ds(off, H)], xh_v)
    pltpu.sync_copy(x_hbm.at[row + 1, pl.ds(off, H)], yh_v)
    pltpu.sync_copy(x_hbm.at[row + 2, pl.ds(off, H)], zh_v)

    inf16 = jnp.full((L,), jnp.inf, jnp.float32)

    def init_j(j, carry):
        dist_v[pl.ds(j * L, L)] = inf16
        return carry

    lax.fori_loop(0, H // L, init_j, 0)
    idx_v[pl.ds(0, L)] = jnp.zeros((L,), jnp.int32)

    lane = lax.iota(jnp.int32, L)

    def step(t, k):
        lx = plsc.load_gather(xs_v, [k])
        ly = plsc.load_gather(ys_v, [k])
        lz = plsc.load_gather(zs_v, [k])

        best0 = jnp.full((L,), -jnp.inf, jnp.float32)
        bidx0 = jnp.full((L,), 2**30, jnp.int32)

        def chunk(s, carry):
            best, bidx = carry
            dx = xh_v[pl.ds(s, L)] - lx
            dy = yh_v[pl.ds(s, L)] - ly
            dz = zh_v[pl.ds(s, L)] - lz
            d = (dx * dx + dz * dz) + dy * dy
            nd = jnp.minimum(dist_v[pl.ds(s, L)], d)
            dist_v[pl.ds(s, L)] = nd
            idx = off + s + lane
            gt = nd > best
            eq = nd == best
            bidx = jnp.where(gt, idx,
                             jnp.where(eq, jnp.minimum(bidx, idx), bidx))
            best = jnp.maximum(best, nd)
            return best, bidx

        best, bidx = plsc.parallel_loop(
            0, H, step=L, unroll=4, carry=(best0, bidx0))(chunk)

        prow = 300 + (t % 2) * 32
        mb_v[...] = best
        mi_v[...] = plsc.bitcast(bidx, jnp.float32)
        pltpu.sync_copy(mb_v, sh_ex.at[cid, prow + sid])
        pltpu.sync_copy(mi_v, sh_ex.at[cid, prow + L + sid])
        plsc.subcore_barrier()
        pltpu.sync_copy(sh_ex.at[cid, prow + (sid ^ 1)], pb_v)
        pltpu.sync_copy(sh_ex.at[cid, prow + L + (sid ^ 1)], pi_v)
        obest = pb_v[...]
        obidx = plsc.bitcast(pi_v[...], jnp.int32)

        gt = obest > best
        eq = obest == best
        bidx = jnp.where(gt, obidx,
                         jnp.where(eq, jnp.minimum(bidx, obidx), bidx))
        best = jnp.maximum(best, obest)

        maxv = jnp.max(best)
        cand = jnp.where(best == maxv, bidx, jnp.int32(2**30))
        knext = jnp.full((L,), jnp.min(cand), jnp.int32)

        @pl.when(half == 0)
        def _():
            plsc.store_scatter(idx_v, [jnp.full((L,), t, jnp.int32)],
                               knext, mask=lane == 0)

        return knext

    lax.fori_loop(1, M, step, jnp.zeros((L,), jnp.int32))

    @pl.when(half == 0)
    def _():
        def gout(j, carry):
            s = j * L
            iv = idx_v[pl.ds(s, L)]
            ox_v[pl.ds(s, L)] = plsc.load_gather(xs_v, [iv])
            oy_v[pl.ds(s, L)] = plsc.load_gather(ys_v, [iv])
            oz_v[pl.ds(s, L)] = plsc.load_gather(zs_v, [iv])
            return carry

        lax.fori_loop(0, M // L, gout, 0)

        pltpu.sync_copy(ox_v, y_hbm.at[row + 0])
        pltpu.sync_copy(oy_v, y_hbm.at[row + 1])
        pltpu.sync_copy(oz_v, y_hbm.at[row + 2])


@jax.jit
def _fps_sc(xr):
    mesh = plsc.VectorSubcoreMesh(core_axis_name="c", subcore_axis_name="s")
    f = functools.partial(
        pl.kernel,
        mesh=mesh,
        compiler_params=pltpu.CompilerParams(needs_layout_passes=False),
        out_type=jax.ShapeDtypeStruct((B * C, M), jnp.float32),
        scratch_types=[
            pltpu.VMEM((N,), jnp.float32),
            pltpu.VMEM((N,), jnp.float32),
            pltpu.VMEM((N,), jnp.float32),
            pltpu.VMEM((H,), jnp.float32),
            pltpu.VMEM((H,), jnp.float32),
            pltpu.VMEM((H,), jnp.float32),
            pltpu.VMEM((H,), jnp.float32),
            pltpu.VMEM((M,), jnp.int32),
            pltpu.VMEM((M,), jnp.float32),
            pltpu.VMEM((M,), jnp.float32),
            pltpu.VMEM((M,), jnp.float32),
            pltpu.VMEM((L,), jnp.float32),
            pltpu.VMEM((L,), jnp.float32),
            pltpu.VMEM((L,), jnp.float32),
            pltpu.VMEM((L,), jnp.float32),
            pltpu.VMEM_SHARED((2, 512, L), jnp.float32),
        ],
    )(_fps_body)
    return f(xr)


def kernel(x):
    xr = x.reshape(B * C, N)
    yr = _fps_sc(xr)
    return yr.reshape(B, C, M)
